# baseline (speedup 1.0000x reference)
import functools

import jax
import jax.numpy as jnp
from jax import lax
from jax.experimental import pallas as pl
from jax.experimental.pallas import tpu as pltpu
from jax.experimental.pallas import tpu_sc as plsc

N = 10000
E = 320000
D = 128
H = 128
G = 64

NC = 2
NS = 16
NW = NC * NS
SPAN = 313
NP = NW * SPAN
EPW = E // NW
SUBCAP = 640
RGN = 32 * SUBCAP
CAPT = 20480 + 256
CG = 256
ZB = 2048
NEG = -3.0e38

_ROWS = 1000


_GDN = lax.GatherDimensionNumbers(
    offset_dims=(), collapsed_slice_dims=(0,), start_index_map=(0,))


def _perm(v, idx):
    return lax.gather(v, idx[:, None], dimension_numbers=_GDN,
                      slice_sizes=(1,), mode=lax.GatherScatterMode.PROMISE_IN_BOUNDS)


def _bin_body(ei_hbm, code_hbm, cnt_hbm, eb, cb, posb, zbuf, cbuf, padp, padc,
              shared, ss0):
    c_ax = lax.axis_index("c")
    s_ax = lax.axis_index("s")
    w = s_ax * NC + c_ax
    region = shared.at[pl.ds(s_ax * RGN, RGN)]

    iota = lax.iota(jnp.int32, 16)
    zeros = jnp.zeros((16,), jnp.int32)
    onev = jnp.full((16,), 1, jnp.int32)
    v512 = jnp.full((16,), 512, jnp.int32)
    spanv = jnp.full((16,), SPAN, jnp.int32)
    subv = jnp.full((16,), SUBCAP, jnp.int32)
    slotcap = jnp.full((16,), SUBCAP - 1, jnp.int32)
    v15 = jnp.full((16,), 15, jnp.int32)
    v16 = jnp.full((16,), 16, jnp.int32)
    vm16 = jnp.full((16,), -16, jnp.int32)
    trash = jnp.full((16,), RGN - 1, jnp.int32)
    rinv = jnp.full((16,), 1.0 / SPAN, jnp.float32)
    half = jnp.full((16,), 0.5, jnp.float32)

    def z(i, _):
        zbuf[pl.ds(i * 16, 16)] = zeros
        return 0

    lax.fori_loop(0, ZB // 16, z, 0)
    for k in range(RGN // ZB):
        pltpu.sync_copy(zbuf, region.at[pl.ds(k * ZB, ZB)])

    pltpu.sync_copy(ei_hbm.at[:, w, :], eb)

    ks = [jnp.full((16,), k, jnp.int32) for k in range(16)]
    laters = [iota > ks[k] for k in range(16)]

    def grp(i, carry):
        off_lo, off_hi = carry
        s = eb[0, pl.ds(i * 16, 16)]
        d = eb[1, pl.ds(i * 16, 16)]
        o = ((d.astype(jnp.float32) + half) * rinv).astype(jnp.int32)
        rank = zeros
        dlo = zeros
        dhi = zeros
        for k in range(16):
            okv = _perm(o, ks[k])
            eq = o == okv
            rank = rank + jnp.where(eq & laters[k], onev, zeros)
            dlo = dlo + jnp.where(iota == okv, onev, zeros)
            dhi = dhi + jnp.where(iota == okv - v16, onev, zeros)
        g1 = _perm(off_lo, jnp.minimum(o, v15))
        g2 = _perm(off_hi, jnp.maximum(o - v16, zeros))
        base = jnp.where(o < v16, g1, g2)
        slot = jnp.minimum(base + rank, slotcap)
        posb[pl.ds(i * 16, 16)] = o * subv + slot
        cb[pl.ds(i * 16, 16)] = s * v512 + (d - o * spanv)
        return off_lo + dlo, off_hi + dhi

    off_lo, off_hi = lax.fori_loop(0, EPW // 16, grp, (zeros, zeros))
    subcapv = jnp.full((16,), SUBCAP - 16, jnp.int32)
    off_lo = jnp.minimum(off_lo, subcapv)
    off_hi = jnp.minimum(off_hi, subcapv)

    pltpu.async_copy(cb, region.at[posb], ss0, add=True)
    pltpu.make_async_copy(cb, region.at[posb], ss0).wait()

    dummyc = jnp.full((16,), SPAN, jnp.int32)
    for j in range(16):
        padc[pl.ds(j * 16, 16)] = dummyc
    for half_idx, offs in ((0, off_lo), (1, off_hi)):
        for oo in range(16):
            o = half_idx * 16 + oo
            cntv = _perm(offs, ks[oo])
            tail = (v16 - (cntv & v15)) & v15
            pos = jnp.where(iota < tail,
                            jnp.full((16,), o * SUBCAP, jnp.int32) + cntv + iota,
                            trash)
            padp[pl.ds(oo * 16, 16)] = pos
        pltpu.sync_copy(padc, region.at[padp], add=True)

    cnt16_lo = (off_lo + v15) & vm16
    cnt16_hi = (off_hi + v15) & vm16
    cbuf[pl.ds(0, 16)] = cnt16_lo
    cbuf[pl.ds(16, 16)] = cnt16_hi
    pltpu.sync_copy(cbuf, cnt_hbm.at[w])
    pltpu.sync_copy(region, code_hbm.at[w])


def _bin_edges(edge_index):
    mesh = plsc.VectorSubcoreMesh(core_axis_name="c", subcore_axis_name="s")
    f = pl.kernel(
        _bin_body,
        mesh=mesh,
        out_type=[
            jax.ShapeDtypeStruct((NW, RGN), jnp.int32),
            jax.ShapeDtypeStruct((NW, 32), jnp.int32),
        ],
        scratch_types=[
            pltpu.VMEM((2, EPW), jnp.int32),
            pltpu.VMEM((EPW,), jnp.int32),
            pltpu.VMEM((EPW,), jnp.int32),
            pltpu.VMEM((ZB,), jnp.int32),
            pltpu.VMEM((32,), jnp.int32),
            pltpu.VMEM((256,), jnp.int32),
            pltpu.VMEM((256,), jnp.int32),
            pltpu.VMEM_SHARED((NS * RGN,), jnp.int32),
            pltpu.SemaphoreType.DMA,
        ],
    )
    return f(edge_index.reshape(2, NW, EPW))


def _segmax_body(h_hbm, code_hbm, cnt_hbm, out_hbm,
                 ls, cv, idxb0, idxb1, rows, agg, sem0, sem1, semL):
    c_ax = lax.axis_index("c")
    s_ax = lax.axis_index("s")
    w = s_ax * NC + c_ax
    sems = (sem0, sem1)

    pltpu.sync_copy(cnt_hbm, cv)
    for wr in range(NW):
        pltpu.async_copy(code_hbm.at[wr, pl.ds(w * SUBCAP, SUBCAP)],
                         ls.at[pl.ds(wr * SUBCAP, SUBCAP)], semL)
    for wr in range(NW):
        pltpu.make_async_copy(code_hbm.at[wr, pl.ds(w * SUBCAP, SUBCAP)],
                              ls.at[pl.ds(wr * SUBCAP, SUBCAP)], semL).wait()

    wlane = w % 16
    hi16 = (w // 16) * 16

    def cnt_of(wr):
        v = cv[wr, pl.ds(hi16, 16)]
        cnt = v[0]
        for j in range(1, 16):
            cnt = jnp.where(wlane == j, v[j], cnt)
        return cnt

    def compact(wr, tot):
        cnt16 = cnt_of(wr)

        def mv(i, _):
            ls[pl.ds(tot + i * 16, 16)] = ls[pl.ds(wr * SUBCAP + i * 16, 16)]
            return 0

        lax.fori_loop(0, cnt16 // 16, mv, 0)
        return tot + cnt16

    tot = 0
    for wr in range(NW):
        tot = compact(wr, tot)

    dummyc = jnp.full((16,), SPAN, jnp.int32)
    for j in range(CG // 16):
        ls[pl.ds(tot + j * 16, 16)] = dummyc
    nch = (tot + CG - 1) // CG

    mask9 = jnp.full((16,), 511, jnp.int32)
    nine = jnp.full((16,), 9, jnp.int32)
    dv = jnp.full((16,), D, jnp.int32)

    idxbs = (idxb0, idxb1)

    def mkidx(c, b):
        def u(i, _):
            idxbs[b][pl.ds(i * 16, 16)] = lax.shift_right_logical(
                ls[pl.ds(c * CG + i * 16, 16)], nine)
            return 0

        lax.fori_loop(0, CG // 16, u, 0)

    def start(c, b):
        mkidx(c, b)
        pltpu.async_copy(h_hbm.at[idxbs[b]], rows.at[b], sems[b])

    def wait(b):
        pltpu.make_async_copy(h_hbm.at[idxbs[b]], rows.at[b],
                              sems[b]).wait()

    negv = jnp.full((16,), NEG, jnp.float32)

    def init(i, _):
        agg[pl.ds(i * 16, 16)] = negv
        return 0

    lax.fori_loop(0, SPAN * D // 16, init, 0)

    lax.cond(0 < nch, lambda: start(0, 0), lambda: None)
    lax.cond(1 < nch, lambda: start(1, 1), lambda: None)

    def chunk(g, _):
        for b in (0, 1):
            c = g * 2 + b

            def do():
                wait(b)

                def group(g2, _):
                    for u in range(4):
                        tg = g2 * 4 + u
                        ldv = (ls[pl.ds(c * CG + tg * 16, 16)] & mask9) * dv
                        for j in range(16):
                            base = pl.multiple_of(ldv[j], 128)
                            e = tg * 16 + j
                            for r in range(D // 16):
                                a = agg[pl.ds(base + r * 16, 16)]
                                v = rows[b, e, pl.ds(r * 16, 16)]
                                agg[pl.ds(base + r * 16, 16)] = jnp.maximum(a, v)
                    return 0

                lax.fori_loop(0, CG // 64, group, 0)
                lax.cond(c + 2 < nch, lambda: start(c + 2, b), lambda: None)

            lax.cond(c < nch, do, lambda: None)
        return 0

    lax.fori_loop(0, (nch + 1) // 2, chunk, 0)

    thr = jnp.full((16,), -1e37, jnp.float32)
    zf = jnp.zeros((16,), jnp.float32)

    def fin(i, _):
        v = agg[pl.ds(i * 16, 16)]
        agg[pl.ds(i * 16, 16)] = jnp.where(v < thr, zf, v)
        return 0

    lax.fori_loop(0, SPAN * D // 16, fin, 0)
    pltpu.sync_copy(agg, out_hbm.at[pl.ds(w * (SPAN + 1) * D, (SPAN + 1) * D)])


def _sc_segmax(h, code_s, counts):
    mesh = plsc.VectorSubcoreMesh(core_axis_name="c", subcore_axis_name="s")
    f = pl.kernel(
        _segmax_body,
        mesh=mesh,
        out_type=jax.ShapeDtypeStruct((NW * (SPAN + 1) * D,), jnp.float32),
        scratch_types=[
            pltpu.VMEM((CAPT,), jnp.int32),
            pltpu.VMEM((NW, 32), jnp.int32),
            pltpu.VMEM((CG,), jnp.int32),
            pltpu.VMEM((CG,), jnp.int32),
            pltpu.VMEM((2, CG, D), jnp.float32),
            pltpu.VMEM(((SPAN + 1) * D,), jnp.float32),
            pltpu.SemaphoreType.DMA,
            pltpu.SemaphoreType.DMA,
            pltpu.SemaphoreType.DMA,
        ],
    )
    flat = f(h, code_s, counts)
    return flat.reshape(NW, SPAN + 1, D)[:, :SPAN].reshape(NP, D)[:N]


def _dense_body(agg_ref, h_ref, Wl_ref, Wr_ref, s_ref, t_ref, o_ref):
    y = jnp.dot(agg_ref[...], Wl_ref[...], preferred_element_type=jnp.float32)
    y += jnp.dot(h_ref[...], Wr_ref[...], preferred_element_type=jnp.float32)
    o_ref[...] = jnp.maximum(y * s_ref[...] + t_ref[...], 0.0)


def _dense_layer(agg, h, Wl, Wr, s, t):
    grid = N // _ROWS
    return pl.pallas_call(
        _dense_body,
        grid=(grid,),
        in_specs=[
            pl.BlockSpec((_ROWS, D), lambda i: (i, 0)),
            pl.BlockSpec((_ROWS, D), lambda i: (i, 0)),
            pl.BlockSpec((D, H), lambda i: (0, 0)),
            pl.BlockSpec((D, H), lambda i: (0, 0)),
            pl.BlockSpec((1, H), lambda i: (0, 0)),
            pl.BlockSpec((1, H), lambda i: (0, 0)),
        ],
        out_specs=pl.BlockSpec((_ROWS, H), lambda i: (i, 0)),
        out_shape=jax.ShapeDtypeStruct((N, H), jnp.float32),
    )(agg, h, Wl, Wr, s, t)


def _pool_body(h_ref, b_ref, Wlin_ref, blin_ref, o_ref):
    bat = b_ref[...]
    gids = jax.lax.broadcasted_iota(jnp.int32, (G, 1), 0)
    onehot = (bat == gids).astype(jnp.float32)
    sums = jnp.dot(onehot, h_ref[...], preferred_element_type=jnp.float32)
    counts = jnp.sum(onehot, axis=1, keepdims=True)
    pooled = sums / jnp.maximum(counts, 1.0)
    o_ref[...] = (
        jnp.dot(pooled, Wlin_ref[...], preferred_element_type=jnp.float32)
        + blin_ref[...]
    )


def _pool_linear(h, batch, Wlin, blin):
    return pl.pallas_call(
        _pool_body,
        in_specs=[
            pl.BlockSpec((N, H), lambda: (0, 0)),
            pl.BlockSpec((1, N), lambda: (0, 0)),
            pl.BlockSpec((H, 1), lambda: (0, 0)),
            pl.BlockSpec((1, 1), lambda: (0, 0)),
        ],
        out_specs=pl.BlockSpec((G, 1), lambda: (0, 0)),
        out_shape=jax.ShapeDtypeStruct((G, 1), jnp.float32),
    )(h, batch.reshape(1, N), Wlin, blin.reshape(1, 1))


def kernel(x, edge_index, batch, Wl1, bl1, Wr1, Wl2, bl2, Wr2, Wl3, bl3, Wr3,
           g1, b1, m1, v1, g2, b2, m2, v2, g3, b3, m3, v3, Wlin, blin):
    eps = 1e-5

    def fold(g, b, m, v, bl):
        s = g / jnp.sqrt(v + eps)
        t = b - m * s + bl * s
        return s.reshape(1, H), t.reshape(1, H)

    s1, t1 = fold(g1, b1, m1, v1, bl1)
    s2, t2 = fold(g2, b2, m2, v2, bl2)
    s3, t3 = fold(g3, b3, m3, v3, bl3)

    code_s, counts = _bin_edges(edge_index)
    h = _dense_layer(_sc_segmax(x, code_s, counts), x, Wl1, Wr1, s1, t1)
    h = _dense_layer(_sc_segmax(h, code_s, counts), h, Wl2, Wr2, s2, t2)
    h = _dense_layer(_sc_segmax(h, code_s, counts), h, Wl3, Wr3, s3, t3)
    return _pool_linear(h, batch, Wlin, blin)


# speedup vs baseline: 1.2668x; 1.2668x over previous
import functools

import jax
import jax.numpy as jnp
from jax import lax
from jax.experimental import pallas as pl
from jax.experimental.pallas import tpu as pltpu
from jax.experimental.pallas import tpu_sc as plsc

N = 10000
E = 320000
D = 128
H = 128
G = 64

NC = 2
NS = 16
NW = NC * NS
SPAN = 313
NP = NW * SPAN
EPW = E // NW
SUBCAP = 640
RGN = 32 * SUBCAP
CAPT = 20480 + 256
CG = 256
ZB = 2048
NEG = -3.0e38

_ROWS = 1000


_GDN = lax.GatherDimensionNumbers(
    offset_dims=(), collapsed_slice_dims=(0,), start_index_map=(0,))


def _perm(v, idx):
    return lax.gather(v, idx[:, None], dimension_numbers=_GDN,
                      slice_sizes=(1,), mode=lax.GatherScatterMode.PROMISE_IN_BOUNDS)


def _bin_body(ei_hbm, code_hbm, cnt_hbm, eb, cb, posb, zbuf, cbuf, padp, padc,
              shared, ss0):
    c_ax = lax.axis_index("c")
    s_ax = lax.axis_index("s")
    w = s_ax * NC + c_ax
    region = shared.at[pl.ds(s_ax * RGN, RGN)]

    iota = lax.iota(jnp.int32, 16)
    zeros = jnp.zeros((16,), jnp.int32)
    onev = jnp.full((16,), 1, jnp.int32)
    v512 = jnp.full((16,), 512, jnp.int32)
    spanv = jnp.full((16,), SPAN, jnp.int32)
    subv = jnp.full((16,), SUBCAP, jnp.int32)
    slotcap = jnp.full((16,), SUBCAP - 1, jnp.int32)
    v15 = jnp.full((16,), 15, jnp.int32)
    v16 = jnp.full((16,), 16, jnp.int32)
    vm16 = jnp.full((16,), -16, jnp.int32)
    trash = jnp.full((16,), RGN - 1, jnp.int32)
    rinv = jnp.full((16,), 1.0 / SPAN, jnp.float32)
    half = jnp.full((16,), 0.5, jnp.float32)

    def z(i, _):
        zbuf[pl.ds(i * 16, 16)] = zeros
        return 0

    lax.fori_loop(0, ZB // 16, z, 0)
    for k in range(RGN // ZB):
        pltpu.sync_copy(zbuf, region.at[pl.ds(k * ZB, ZB)])

    pltpu.sync_copy(ei_hbm.at[:, w, :], eb)

    ks = [jnp.full((16,), k, jnp.int32) for k in range(16)]
    laters = [iota > ks[k] for k in range(16)]

    def grp(i, carry):
        off_lo, off_hi = carry
        s = eb[0, pl.ds(i * 16, 16)]
        d = eb[1, pl.ds(i * 16, 16)]
        o = ((d.astype(jnp.float32) + half) * rinv).astype(jnp.int32)
        rank = zeros
        dlo = zeros
        dhi = zeros
        for k in range(16):
            okv = _perm(o, ks[k])
            eq = o == okv
            rank = rank + jnp.where(eq & laters[k], onev, zeros)
            dlo = dlo + jnp.where(iota == okv, onev, zeros)
            dhi = dhi + jnp.where(iota == okv - v16, onev, zeros)
        g1 = _perm(off_lo, jnp.minimum(o, v15))
        g2 = _perm(off_hi, jnp.maximum(o - v16, zeros))
        base = jnp.where(o < v16, g1, g2)
        slot = jnp.minimum(base + rank, slotcap)
        posb[pl.ds(i * 16, 16)] = o * subv + slot
        cb[pl.ds(i * 16, 16)] = s * v512 + (d - o * spanv)
        return off_lo + dlo, off_hi + dhi

    off_lo, off_hi = lax.fori_loop(0, EPW // 16, grp, (zeros, zeros))
    subcapv = jnp.full((16,), SUBCAP - 16, jnp.int32)
    off_lo = jnp.minimum(off_lo, subcapv)
    off_hi = jnp.minimum(off_hi, subcapv)

    pltpu.async_copy(cb, region.at[posb], ss0, add=True)
    pltpu.make_async_copy(cb, region.at[posb], ss0).wait()

    dummyc = jnp.full((16,), SPAN, jnp.int32)
    for j in range(16):
        padc[pl.ds(j * 16, 16)] = dummyc
    for half_idx, offs in ((0, off_lo), (1, off_hi)):
        for oo in range(16):
            o = half_idx * 16 + oo
            cntv = _perm(offs, ks[oo])
            tail = (v16 - (cntv & v15)) & v15
            pos = jnp.where(iota < tail,
                            jnp.full((16,), o * SUBCAP, jnp.int32) + cntv + iota,
                            trash)
            padp[pl.ds(oo * 16, 16)] = pos
        pltpu.sync_copy(padc, region.at[padp], add=True)

    cnt16_lo = (off_lo + v15) & vm16
    cnt16_hi = (off_hi + v15) & vm16
    cbuf[pl.ds(0, 16)] = cnt16_lo
    cbuf[pl.ds(16, 16)] = cnt16_hi
    pltpu.sync_copy(cbuf, cnt_hbm.at[w])
    pltpu.sync_copy(region, code_hbm.at[w])


def _bin_edges(edge_index):
    mesh = plsc.VectorSubcoreMesh(core_axis_name="c", subcore_axis_name="s")
    f = pl.kernel(
        _bin_body,
        mesh=mesh,
        out_type=[
            jax.ShapeDtypeStruct((NW, RGN), jnp.int32),
            jax.ShapeDtypeStruct((NW, 32), jnp.int32),
        ],
        scratch_types=[
            pltpu.VMEM((2, EPW), jnp.int32),
            pltpu.VMEM((EPW,), jnp.int32),
            pltpu.VMEM((EPW,), jnp.int32),
            pltpu.VMEM((ZB,), jnp.int32),
            pltpu.VMEM((32,), jnp.int32),
            pltpu.VMEM((256,), jnp.int32),
            pltpu.VMEM((256,), jnp.int32),
            pltpu.VMEM_SHARED((NS * RGN,), jnp.int32),
            pltpu.SemaphoreType.DMA,
        ],
    )
    return f(edge_index.reshape(2, NW, EPW))


def _segmax_body(h_hbm, code_hbm, cnt_hbm, out_hbm,
                 ls, cv, idxb0, idxb1, rows, agg, sem0, sem1, semL):
    c_ax = lax.axis_index("c")
    s_ax = lax.axis_index("s")
    w = s_ax * NC + c_ax
    sems = (sem0, sem1)

    pltpu.sync_copy(cnt_hbm, cv)
    for wr in range(NW):
        pltpu.async_copy(code_hbm.at[wr, pl.ds(w * SUBCAP, SUBCAP)],
                         ls.at[pl.ds(wr * SUBCAP, SUBCAP)], semL)
    for wr in range(NW):
        pltpu.make_async_copy(code_hbm.at[wr, pl.ds(w * SUBCAP, SUBCAP)],
                              ls.at[pl.ds(wr * SUBCAP, SUBCAP)], semL).wait()

    wlane = w % 16
    hi16 = (w // 16) * 16

    def cnt_of(wr):
        v = cv[wr, pl.ds(hi16, 16)]
        cnt = v[0]
        for j in range(1, 16):
            cnt = jnp.where(wlane == j, v[j], cnt)
        return cnt

    def compact(wr, tot):
        cnt16 = cnt_of(wr)

        def mv(i, _):
            ls[pl.ds(tot + i * 16, 16)] = ls[pl.ds(wr * SUBCAP + i * 16, 16)]
            return 0

        lax.fori_loop(0, cnt16 // 16, mv, 0)
        return tot + cnt16

    tot = 0
    for wr in range(NW):
        tot = compact(wr, tot)

    dummyc = jnp.full((16,), SPAN, jnp.int32)
    for j in range(CG // 16):
        ls[pl.ds(tot + j * 16, 16)] = dummyc
    nch = (tot + CG - 1) // CG

    mask9 = jnp.full((16,), 511, jnp.int32)
    nine = jnp.full((16,), 9, jnp.int32)
    dv = jnp.full((16,), D, jnp.int32)

    idxbs = (idxb0, idxb1)

    def mkidx(c, b):
        def u(i, _):
            idxbs[b][pl.ds(i * 16, 16)] = lax.shift_right_logical(
                ls[pl.ds(c * CG + i * 16, 16)], nine)
            return 0

        lax.fori_loop(0, CG // 16, u, 0)

    def start(c, b):
        mkidx(c, b)
        pltpu.async_copy(h_hbm.at[idxbs[b]], rows.at[b], sems[b])

    def wait(b):
        pltpu.make_async_copy(h_hbm.at[idxbs[b]], rows.at[b],
                              sems[b]).wait()

    negv = jnp.full((16,), NEG, jnp.float32)

    def init(i, _):
        agg[pl.ds(i * 16, 16)] = negv
        return 0

    lax.fori_loop(0, SPAN * D // 16, init, 0)

    lax.cond(0 < nch, lambda: start(0, 0), lambda: None)
    lax.cond(1 < nch, lambda: start(1, 1), lambda: None)

    def chunk(g, _):
        for b in (0, 1):
            c = g * 2 + b

            def do():
                wait(b)

                def group(g2, _):
                    for u in range(2):
                        tg = g2 * 2 + u
                        ldv = (ls[pl.ds(c * CG + tg * 16, 16)] & mask9) * dv
                        for j in range(16):
                            base = pl.multiple_of(ldv[j], 128)
                            e = tg * 16 + j
                            for r in range(D // 16):
                                a = agg[pl.ds(base + r * 16, 16)]
                                v = rows[b, e, pl.ds(r * 16, 16)]
                                agg[pl.ds(base + r * 16, 16)] = jnp.maximum(a, v)
                    return 0

                lax.fori_loop(0, CG // 32, group, 0)
                lax.cond(c + 2 < nch, lambda: start(c + 2, b), lambda: None)

            lax.cond(c < nch, do, lambda: None)
        return 0

    lax.fori_loop(0, (nch + 1) // 2, chunk, 0)

    thr = jnp.full((16,), -1e37, jnp.float32)
    zf = jnp.zeros((16,), jnp.float32)

    def fin(i, _):
        v = agg[pl.ds(i * 16, 16)]
        agg[pl.ds(i * 16, 16)] = jnp.where(v < thr, zf, v)
        return 0

    lax.fori_loop(0, SPAN * D // 16, fin, 0)
    pltpu.sync_copy(agg, out_hbm.at[pl.ds(w * (SPAN + 1) * D, (SPAN + 1) * D)])


def _sc_segmax(h, code_s, counts):
    mesh = plsc.VectorSubcoreMesh(core_axis_name="c", subcore_axis_name="s")
    f = pl.kernel(
        _segmax_body,
        mesh=mesh,
        out_type=jax.ShapeDtypeStruct((NW * (SPAN + 1) * D,), jnp.float32),
        scratch_types=[
            pltpu.VMEM((CAPT,), jnp.int32),
            pltpu.VMEM((NW, 32), jnp.int32),
            pltpu.VMEM((CG,), jnp.int32),
            pltpu.VMEM((CG,), jnp.int32),
            pltpu.VMEM((2, CG, D), jnp.float32),
            pltpu.VMEM(((SPAN + 1) * D,), jnp.float32),
            pltpu.SemaphoreType.DMA,
            pltpu.SemaphoreType.DMA,
            pltpu.SemaphoreType.DMA,
        ],
    )
    flat = f(h, code_s, counts)
    return flat.reshape(NW, SPAN + 1, D)[:, :SPAN].reshape(NP, D)[:N]


def _dense_body(agg_ref, h_ref, Wl_ref, Wr_ref, s_ref, t_ref, o_ref):
    y = jnp.dot(agg_ref[...], Wl_ref[...], preferred_element_type=jnp.float32)
    y += jnp.dot(h_ref[...], Wr_ref[...], preferred_element_type=jnp.float32)
    o_ref[...] = jnp.maximum(y * s_ref[...] + t_ref[...], 0.0)


def _dense_layer(agg, h, Wl, Wr, s, t):
    grid = N // _ROWS
    return pl.pallas_call(
        _dense_body,
        grid=(grid,),
        in_specs=[
            pl.BlockSpec((_ROWS, D), lambda i: (i, 0)),
            pl.BlockSpec((_ROWS, D), lambda i: (i, 0)),
            pl.BlockSpec((D, H), lambda i: (0, 0)),
            pl.BlockSpec((D, H), lambda i: (0, 0)),
            pl.BlockSpec((1, H), lambda i: (0, 0)),
            pl.BlockSpec((1, H), lambda i: (0, 0)),
        ],
        out_specs=pl.BlockSpec((_ROWS, H), lambda i: (i, 0)),
        out_shape=jax.ShapeDtypeStruct((N, H), jnp.float32),
    )(agg, h, Wl, Wr, s, t)


def _pool_body(h_ref, b_ref, Wlin_ref, blin_ref, o_ref):
    bat = b_ref[...]
    gids = jax.lax.broadcasted_iota(jnp.int32, (G, 1), 0)
    onehot = (bat == gids).astype(jnp.float32)
    sums = jnp.dot(onehot, h_ref[...], preferred_element_type=jnp.float32)
    counts = jnp.sum(onehot, axis=1, keepdims=True)
    pooled = sums / jnp.maximum(counts, 1.0)
    o_ref[...] = (
        jnp.dot(pooled, Wlin_ref[...], preferred_element_type=jnp.float32)
        + blin_ref[...]
    )


def _pool_linear(h, batch, Wlin, blin):
    return pl.pallas_call(
        _pool_body,
        in_specs=[
            pl.BlockSpec((N, H), lambda: (0, 0)),
            pl.BlockSpec((1, N), lambda: (0, 0)),
            pl.BlockSpec((H, 1), lambda: (0, 0)),
            pl.BlockSpec((1, 1), lambda: (0, 0)),
        ],
        out_specs=pl.BlockSpec((G, 1), lambda: (0, 0)),
        out_shape=jax.ShapeDtypeStruct((G, 1), jnp.float32),
    )(h, batch.reshape(1, N), Wlin, blin.reshape(1, 1))


def kernel(x, edge_index, batch, Wl1, bl1, Wr1, Wl2, bl2, Wr2, Wl3, bl3, Wr3,
           g1, b1, m1, v1, g2, b2, m2, v2, g3, b3, m3, v3, Wlin, blin):
    eps = 1e-5

    def fold(g, b, m, v, bl):
        s = g / jnp.sqrt(v + eps)
        t = b - m * s + bl * s
        return s.reshape(1, H), t.reshape(1, H)

    s1, t1 = fold(g1, b1, m1, v1, bl1)
    s2, t2 = fold(g2, b2, m2, v2, bl2)
    s3, t3 = fold(g3, b3, m3, v3, bl3)

    code_s, counts = _bin_edges(edge_index)
    h = _dense_layer(_sc_segmax(x, code_s, counts), x, Wl1, Wr1, s1, t1)
    h = _dense_layer(_sc_segmax(h, code_s, counts), h, Wl2, Wr2, s2, t2)
    h = _dense_layer(_sc_segmax(h, code_s, counts), h, Wl3, Wr3, s3, t3)
    return _pool_linear(h, batch, Wlin, blin)
